# Initial kernel scaffold; baseline (speedup 1.0000x reference)
import jax
import jax.numpy as jnp
from jax.experimental import pallas as pl


def kernel(activity_ids, embed, W1, a_src1, a_dst1, b1, W2, a_src2, a_dst2, b2, Wr1, br1, Wr2, br2, We, be, Wo, bo):
    raise NotImplementedError("write your pallas kernel here")


# speedup vs baseline: 44.2323x; 44.2323x over previous
import functools

import jax
import jax.numpy as jnp
from jax import lax
from jax.experimental import pallas as pl
from jax.experimental.pallas import tpu as pltpu
from jax.experimental.pallas import tpu_sc as plsc

A = 21
D = 128
B = 4096
S = 200
T = S - 1
AP = 32
GH = 1024
RH = 448


def _lut_body(o_ref):
    r = lax.broadcasted_iota(jnp.int32, (2, 128), 0)
    c = lax.broadcasted_iota(jnp.int32, (2, 128), 1)
    n = (r * 128 + c).astype(jnp.float32)
    o_ref[...] = (-1.0 / float(T)) * jnp.log(n / float(T) + 1e-10)


def _lut_call():
    return pl.pallas_call(
        _lut_body, out_shape=jax.ShapeDtypeStruct((2, 128), jnp.float32)
    )()


def _sc_body(rows_per, nc, ids_hbm, lut_hbm, me_hbm, gh_hbm,
             ids_v, lut_v, rowhist_v, tothist_v, me_v):
    cid = lax.axis_index("c")
    sid = lax.axis_index("s")
    wid = sid * nc + cid
    base = wid * rows_per

    pltpu.sync_copy(ids_hbm.at[pl.ds(base * S, rows_per * S)], ids_v)
    pltpu.sync_copy(lut_hbm, lut_v)

    zeros16 = jnp.zeros((16,), jnp.float32)
    ones16 = jnp.ones((16,), jnp.float32)
    iota16 = lax.iota(jnp.int32, 16)
    nchunk = (T + 15) // 16

    for k in range(RH // 16):
        rowhist_v[pl.ds(16 * k, 16)] = zeros16
    for k in range(GH // 16):
        tothist_v[pl.ds(16 * k, 16)] = zeros16

    def load_pair(i, t):
        rbase = i * S
        col = t * 16 + iota16
        if t == nchunk - 1:
            m = col < T
            ca = jnp.minimum(col, T - 1)
            cb = jnp.minimum(col + 1, T)
        else:
            m = None
            ca = col
            cb = col + 1
        a = plsc.load_gather(ids_v, [rbase + ca], mask=m)
        b = plsc.load_gather(ids_v, [rbase + cb], mask=m)
        return a, b, m

    def row_body(i, carry):
        for t in range(nchunk):
            a, b, m = load_pair(i, t)
            code = a * A + b
            plsc.addupdate_scatter(rowhist_v, [code], ones16, mask=m)
            gcode = b * AP + a
            plsc.addupdate_scatter(tothist_v, [gcode], ones16, mask=m)
        acc = zeros16
        for t in range(nchunk):
            a, b, m = load_pair(i, t)
            code = a * A + b
            cnt = plsc.load_gather(rowhist_v, [code], mask=m)
            lidx = cnt.astype(jnp.int32)
            lv = plsc.load_gather(lut_v, [lidx], mask=m)
            if m is not None:
                lv = jnp.where(m, lv, 0.0)
            acc = acc + lv
        for k in range(RH // 16):
            rowhist_v[pl.ds(16 * k, 16)] = zeros16
        me_i = jnp.sum(acc)
        plsc.store_scatter(me_v, [jnp.full((16,), i, jnp.int32)],
                           jnp.full((16,), me_i, jnp.float32),
                           mask=iota16 == 0)
        return carry

    lax.fori_loop(0, rows_per, row_body, 0)

    pltpu.sync_copy(me_v, me_hbm.at[pl.ds(base, rows_per)])
    pltpu.sync_copy(tothist_v, gh_hbm.at[wid])


def _sc_call(ids, lut):
    info = plsc.get_sparse_core_info()
    nc, ns = info.num_cores, info.num_subcores
    nw = nc * ns
    rows_per = B // nw
    mesh = plsc.VectorSubcoreMesh(core_axis_name="c", subcore_axis_name="s")
    fn = pl.kernel(
        functools.partial(_sc_body, rows_per, nc),
        out_type=[
            jax.ShapeDtypeStruct((B,), jnp.float32),
            jax.ShapeDtypeStruct((nw, GH), jnp.float32),
        ],
        mesh=mesh,
        compiler_params=pltpu.CompilerParams(needs_layout_passes=False),
        scratch_types=[
            pltpu.VMEM((rows_per * S,), jnp.int32),
            pltpu.VMEM((256,), jnp.float32),
            pltpu.VMEM((RH,), jnp.float32),
            pltpu.VMEM((GH,), jnp.float32),
            pltpu.VMEM((rows_per,), jnp.float32),
        ],
    )
    return fn(ids.reshape(B * S), lut)


def _graph_body(nw, hists_ref, me_ref, embed_ref, w1_ref, msrc1_ref,
                mdst1_ref, b1_ref, w2_ref, asrc2_ref, adst2_ref, b2_ref,
                wr1_ref, br1_ref, wr2_ref, br2_ref, we_ref, be_ref,
                wotop_ref, wobot_ref, bo_ref, out_ref):
    f32 = jnp.float32

    gh = hists_ref[0]
    for n in range(1, nw):
        gh = gh + hists_ref[n]
    ri = lax.broadcasted_iota(jnp.int32, (AP, AP), 0)
    ci = lax.broadcasted_iota(jnp.int32, (AP, AP), 1)
    maskt = (gh > 0.0) | (ri == ci)

    def gat_attention(adst_col, asrc_row, hsrc):
        e = adst_col + asrc_row
        e = jnp.where(e >= 0.0, e, 0.2 * e)
        e = jnp.where(maskt, e, -1e9)
        m = jnp.max(e, axis=1, keepdims=True)
        p = jnp.exp(e - m)
        att = p / jnp.sum(p, axis=1, keepdims=True)
        return jnp.dot(att, hsrc, preferred_element_type=f32)

    dn_t = (((1,), (1,)), ((), ()))

    x = embed_ref[...]
    h1 = jnp.dot(x, w1_ref[...], preferred_element_type=f32)
    asrc1 = lax.dot_general(msrc1_ref[...], h1, dn_t,
                            preferred_element_type=f32)
    adst1 = lax.dot_general(h1, mdst1_ref[...], dn_t,
                            preferred_element_type=f32)
    heads = []
    for h in range(4):
        heads.append(gat_attention(adst1[:, h:h + 1], asrc1[h:h + 1, :],
                                   h1[:, 64 * h:64 * h + 64]))
    h1o = jnp.concatenate(heads, axis=1) + b1_ref[...]
    h1o = jnp.where(h1o > 0.0, h1o, jnp.exp(h1o) - 1.0)

    h2 = jnp.dot(h1o, w2_ref[...], preferred_element_type=f32)
    asrc2 = lax.dot_general(asrc2_ref[...], h2, dn_t,
                            preferred_element_type=f32)
    adst2 = lax.dot_general(h2, adst2_ref[...], dn_t,
                            preferred_element_type=f32)
    h2o = gat_attention(adst2[:, 0:1], asrc2[0:1, :], h2) + b2_ref[...]

    g1 = jnp.dot(h2o[0:1, :], wr1_ref[0:D, :], preferred_element_type=f32)
    for n in range(1, AP):
        g1 = g1 + jnp.dot(h2o[n:n + 1, :], wr1_ref[D * n:D * n + D, :],
                          preferred_element_type=f32)
    g1 = g1 + br1_ref[...]
    g1 = jnp.maximum(g1, 0.0)
    g2 = jnp.dot(g1, wr2_ref[...], preferred_element_type=f32) + br2_ref[...]

    const = (jnp.dot(g2, wotop_ref[...], preferred_element_type=f32)
             + jnp.dot(be_ref[0:1, 0:32], wobot_ref[...],
                       preferred_element_type=f32)
             + bo_ref[...])
    v = jnp.dot(we_ref[0:1, 0:32], wobot_ref[...],
                preferred_element_type=f32)

    out_ref[...] = const + me_ref[...] * v


def _graph_call(hists, me2, embed_p, w1, msrc1, mdst1, b1r, w2, asrc2p,
                adst2p, b2r, wr1p, br1r, wr2, br2r, wep, bep, wotop, wobot,
                bor):
    nw = hists.shape[0]
    return pl.pallas_call(
        functools.partial(_graph_body, nw),
        out_shape=jax.ShapeDtypeStruct((B, D), jnp.float32),
    )(hists, me2, embed_p, w1, msrc1, mdst1, b1r, w2, asrc2p, adst2p, b2r,
      wr1p, br1r, wr2, br2r, wep, bep, wotop, wobot, bor)


def kernel(activity_ids, embed, W1, a_src1, a_dst1, b1, W2, a_src2, a_dst2,
           b2, Wr1, br1, Wr2, br2, We, be, Wo, bo):
    f32 = jnp.float32

    lut = _lut_call().reshape(256)
    me, hists = _sc_call(activity_ids, lut)

    embed_p = jnp.zeros((AP, D), f32).at[:A].set(embed)
    eye4 = jnp.eye(4, dtype=f32)
    msrc1 = jnp.zeros((8, 256), f32).at[:4].set(
        (eye4[:, :, None] * a_src1[None, :, :]).reshape(4, 256))
    mdst1 = jnp.zeros((8, 256), f32).at[:4].set(
        (eye4[:, :, None] * a_dst1[None, :, :]).reshape(4, 256))
    asrc2p = jnp.zeros((8, D), f32).at[0].set(a_src2[0])
    adst2p = jnp.zeros((8, D), f32).at[0].set(a_dst2[0])
    wr1p = jnp.zeros((AP, D, 2 * D), f32).at[:A].set(
        Wr1.reshape(A, D, 2 * D)).reshape(AP * D, 2 * D)
    wep = jnp.zeros((8, D), f32).at[0, :32].set(We[0])
    bep = jnp.zeros((8, D), f32).at[0, :32].set(be)
    hists3 = hists.reshape(hists.shape[0], AP, AP)

    h_mp = _graph_call(
        hists3, me.reshape(B, 1), embed_p, W1, msrc1, mdst1,
        b1.reshape(1, 256), W2, asrc2p, adst2p, b2.reshape(1, D), wr1p,
        br1.reshape(1, 256), Wr2, br2.reshape(1, D), wep, bep, Wo[:D],
        Wo[D:], bo.reshape(1, D))
    return h_mp, me


# speedup vs baseline: 47.4440x; 1.0726x over previous
import functools

import jax
import jax.numpy as jnp
from jax import lax
from jax.experimental import pallas as pl
from jax.experimental.pallas import tpu as pltpu
from jax.experimental.pallas import tpu_sc as plsc

A = 21
D = 128
B = 4096
S = 200
T = S - 1
GH = 1024
RH = 448
NCHUNK = (T + 15) // 16


def _sc_body(rows_per, nc, ids_hbm, me_hbm, gh_hbm,
             ids_v, lut_v, codes_v, rowhist_v, tothist_v, me_v):
    cid = lax.axis_index("c")
    sid = lax.axis_index("s")
    wid = sid * nc + cid
    base = wid * rows_per

    pltpu.sync_copy(ids_hbm.at[pl.ds(base * S, rows_per * S)],
                    ids_v.at[pl.ds(0, rows_per * S)])

    zeros16i = jnp.zeros((16,), jnp.int32)
    ones16i = jnp.ones((16,), jnp.int32)
    ones16f = jnp.ones((16,), jnp.float32)
    iota16 = lax.iota(jnp.int32, 16)

    for k in range(256 // 16):
        x = (k * 16 + iota16).astype(jnp.float32) * (1.0 / T) + 1e-10
        bits = plsc.bitcast(x, jnp.int32)
        ex = (bits >> 23) - 127
        mant = plsc.bitcast((bits & 0x7FFFFF) | 0x3F800000, jnp.float32)
        adj = mant >= 1.4142135
        mant = jnp.where(adj, 0.5 * mant, mant)
        ef = ex.astype(jnp.float32) + jnp.where(adj, 1.0, 0.0)
        t = (mant - 1.0) / (mant + 1.0)
        t2 = t * t
        lnm = t * (2.0 + t2 * (2.0 / 3.0 + t2 * (2.0 / 5.0 + t2 * (2.0 / 7.0))))
        lnx = ef * 0.69314718 + lnm
        lut_v[pl.ds(16 * k, 16)] = lnx * (-1.0 / T)

    for k in range(RH // 16):
        rowhist_v[pl.ds(16 * k, 16)] = zeros16i
    for k in range(GH // 16):
        tothist_v[pl.ds(16 * k, 16)] = jnp.zeros((16,), jnp.float32)

    mlast = iota16 < (T - 16 * (NCHUNK - 1))

    def row_body(i, carry):
        rbase = i * S
        for t in range(NCHUNK):
            m = mlast if t == NCHUNK - 1 else None
            a = ids_v[pl.ds(rbase + 16 * t, 16)]
            b = ids_v[pl.ds(rbase + 16 * t + 1, 16)]
            code = a * A + b
            codes_v[pl.ds(16 * t, 16)] = code
            plsc.addupdate_scatter(rowhist_v, [code], ones16i, mask=m)
            gcode = (b << 5) + a
            plsc.addupdate_scatter(tothist_v, [gcode], ones16f, mask=m)
        acc = jnp.zeros((16,), jnp.float32)
        for t in range(NCHUNK):
            m = mlast if t == NCHUNK - 1 else None
            code = codes_v[pl.ds(16 * t, 16)]
            cnt = plsc.load_gather(rowhist_v, [code], mask=m)
            lv = plsc.load_gather(lut_v, [cnt], mask=m)
            if m is not None:
                lv = jnp.where(m, lv, 0.0)
            acc = acc + lv
        for k in range(RH // 16):
            rowhist_v[pl.ds(16 * k, 16)] = zeros16i
        me_i = jnp.sum(acc)
        plsc.store_scatter(me_v, [jnp.full((16,), i, jnp.int32)],
                           jnp.full((16,), me_i, jnp.float32),
                           mask=iota16 == 0)
        return carry

    lax.fori_loop(0, rows_per, row_body, 0)

    pltpu.sync_copy(me_v, me_hbm.at[pl.ds(base, rows_per)])
    pltpu.sync_copy(tothist_v, gh_hbm.at[wid])


def _sc_call(ids):
    info = plsc.get_sparse_core_info()
    nc, ns = info.num_cores, info.num_subcores
    nw = nc * ns
    rows_per = B // nw
    mesh = plsc.VectorSubcoreMesh(core_axis_name="c", subcore_axis_name="s")
    fn = pl.kernel(
        functools.partial(_sc_body, rows_per, nc),
        out_type=[
            jax.ShapeDtypeStruct((B,), jnp.float32),
            jax.ShapeDtypeStruct((nw, GH), jnp.float32),
        ],
        mesh=mesh,
        compiler_params=pltpu.CompilerParams(needs_layout_passes=False),
        scratch_types=[
            pltpu.VMEM((rows_per * S + 16,), jnp.int32),
            pltpu.VMEM((256,), jnp.float32),
            pltpu.VMEM((16 * NCHUNK,), jnp.int32),
            pltpu.VMEM((RH,), jnp.int32),
            pltpu.VMEM((GH,), jnp.float32),
            pltpu.VMEM((rows_per,), jnp.float32),
        ],
    )
    return fn(ids.reshape(B * S))


def _graph_body(nw, hists_ref, me_ref, embed_ref, w1_ref, asrc1_ref,
                adst1_ref, b1_ref, w2_ref, asrc2_ref, adst2_ref, b2_ref,
                wr1_ref, br1_ref, wr2_ref, br2_ref, we_ref, be_ref,
                wotop_ref, wobot_ref, bo_ref, out_ref):
    f32 = jnp.float32

    gh = hists_ref[0]
    for n in range(1, nw):
        gh = gh + hists_ref[n]
    ghd = gh[0:A, 0:A]
    ri = lax.broadcasted_iota(jnp.int32, (A, A), 0)
    ci = lax.broadcasted_iota(jnp.int32, (A, A), 1)
    maskt = (ghd > 0.0) | (ri == ci)

    def gat_attention(adst_col, asrc_row, hsrc):
        e = adst_col + asrc_row
        e = jnp.where(e >= 0.0, e, 0.2 * e)
        e = jnp.where(maskt, e, -1e9)
        m = jnp.max(e, axis=1, keepdims=True)
        p = jnp.exp(e - m)
        att = p / jnp.sum(p, axis=1, keepdims=True)
        return jnp.dot(att, hsrc, preferred_element_type=f32)

    dn_t = (((1,), (1,)), ((), ()))

    x = embed_ref[...]
    h1 = jnp.dot(x, w1_ref[...], preferred_element_type=f32)
    heads = []
    for h in range(4):
        hh = h1[:, 64 * h:64 * h + 64]
        asrc = lax.dot_general(asrc1_ref[h:h + 1, :], hh, dn_t,
                               preferred_element_type=f32)
        adst = lax.dot_general(hh, adst1_ref[h:h + 1, :], dn_t,
                               preferred_element_type=f32)
        heads.append(gat_attention(adst, asrc, hh))
    h1o = jnp.concatenate(heads, axis=1) + b1_ref[...]
    h1o = jnp.where(h1o > 0.0, h1o, jnp.exp(h1o) - 1.0)

    h2 = jnp.dot(h1o, w2_ref[...], preferred_element_type=f32)
    asrc2 = lax.dot_general(asrc2_ref[...], h2, dn_t,
                            preferred_element_type=f32)
    adst2 = lax.dot_general(h2, adst2_ref[...], dn_t,
                            preferred_element_type=f32)
    h2o = gat_attention(adst2, asrc2, h2) + b2_ref[...]

    g1 = jnp.dot(h2o[0:1, :], wr1_ref[0:D, :], preferred_element_type=f32)
    for n in range(1, A):
        g1 = g1 + jnp.dot(h2o[n:n + 1, :], wr1_ref[D * n:D * n + D, :],
                          preferred_element_type=f32)
    g1 = g1 + br1_ref[...]
    g1 = jnp.maximum(g1, 0.0)
    g2 = jnp.dot(g1, wr2_ref[...], preferred_element_type=f32) + br2_ref[...]

    const = (jnp.dot(g2, wotop_ref[...], preferred_element_type=f32)
             + jnp.dot(be_ref[...], wobot_ref[...],
                       preferred_element_type=f32)
             + bo_ref[...])
    v = jnp.dot(we_ref[...], wobot_ref[...],
                preferred_element_type=f32)

    out_ref[...] = const + me_ref[...] * v


def _graph_call(hists3, me2, embed, w1, a_src1, a_dst1, b1r, w2, a_src2,
                a_dst2, b2r, wr1, br1r, wr2, br2r, we, ber, wotop, wobot,
                bor):
    nw = hists3.shape[0]
    return pl.pallas_call(
        functools.partial(_graph_body, nw),
        out_shape=jax.ShapeDtypeStruct((B, D), jnp.float32),
    )(hists3, me2, embed, w1, a_src1, a_dst1, b1r, w2, a_src2, a_dst2, b2r,
      wr1, br1r, wr2, br2r, we, ber, wotop, wobot, bor)


def kernel(activity_ids, embed, W1, a_src1, a_dst1, b1, W2, a_src2, a_dst2,
           b2, Wr1, br1, Wr2, br2, We, be, Wo, bo):
    me, hists = _sc_call(activity_ids)
    h_mp = _graph_call(
        hists.reshape(hists.shape[0], 32, 32), me.reshape(B, 1), embed, W1,
        a_src1, a_dst1, b1.reshape(1, 256), W2, a_src2, a_dst2,
        b2.reshape(1, D), Wr1, br1.reshape(1, 256), Wr2, br2.reshape(1, D),
        We, be.reshape(1, 32), Wo[:D], Wo[D:], bo.reshape(1, D))
    return h_mp, me


# speedup vs baseline: 52.8171x; 1.1133x over previous
import functools

import jax
import jax.numpy as jnp
from jax import lax
from jax.experimental import pallas as pl
from jax.experimental.pallas import tpu as pltpu
from jax.experimental.pallas import tpu_sc as plsc

A = 21
D = 128
B = 4096
S = 200
T = S - 1
GH = 1024
RH = 448
NCHUNK = (T + 15) // 16


def _sc_body(rows_per, nc, ids_hbm, me_hbm, gh_hbm,
             ids_v, lut_v, codes_v, rowhist_v, tothist_v, me_v):
    cid = lax.axis_index("c")
    sid = lax.axis_index("s")
    wid = sid * nc + cid
    base = wid * rows_per

    pltpu.sync_copy(ids_hbm.at[pl.ds(base, rows_per)], ids_v)

    zeros16i = jnp.zeros((16,), jnp.int32)
    ones16i = jnp.ones((16,), jnp.int32)
    ones16f = jnp.ones((16,), jnp.float32)
    iota16 = lax.iota(jnp.int32, 16)

    for k in range(256 // 16):
        x = (k * 16 + iota16).astype(jnp.float32) * (1.0 / T) + 1e-10
        bits = plsc.bitcast(x, jnp.int32)
        ex = (bits >> 23) - 127
        mant = plsc.bitcast((bits & 0x7FFFFF) | 0x3F800000, jnp.float32)
        adj = mant >= 1.4142135
        mant = jnp.where(adj, 0.5 * mant, mant)
        ef = ex.astype(jnp.float32) + jnp.where(adj, 1.0, 0.0)
        t = (mant - 1.0) / (mant + 1.0)
        t2 = t * t
        lnm = t * (2.0 + t2 * (2.0 / 3.0 + t2 * (2.0 / 5.0 + t2 * (2.0 / 7.0))))
        lnx = ef * 0.69314718 + lnm
        lut_v[pl.ds(16 * k, 16)] = lnx * (-1.0 / T)

    for k in range(RH // 16):
        rowhist_v[pl.ds(16 * k, 16)] = zeros16i
    for k in range(GH // 16):
        tothist_v[pl.ds(16 * k, 16)] = jnp.zeros((16,), jnp.float32)

    mlast = iota16 < (T - 16 * (NCHUNK - 1))

    def row_body(i, carry):
        rowi = jnp.full((16,), i, jnp.int32)
        for t in range(NCHUNK):
            col = 16 * t + iota16
            if t == NCHUNK - 1:
                m = mlast
                ca = jnp.minimum(col, T - 1)
                cb = jnp.minimum(col + 1, T)
            else:
                m = None
                ca = col
                cb = col + 1
            a = plsc.load_gather(ids_v, [rowi, ca], mask=m)
            b = plsc.load_gather(ids_v, [rowi, cb], mask=m)
            code = a * A + b
            codes_v[pl.ds(16 * t, 16)] = code
            plsc.addupdate_scatter(rowhist_v, [code], ones16i, mask=m)
            gcode = (b << 5) + a
            plsc.addupdate_scatter(tothist_v, [gcode], ones16f, mask=m)
        acc = jnp.zeros((16,), jnp.float32)
        for t in range(NCHUNK):
            m = mlast if t == NCHUNK - 1 else None
            code = codes_v[pl.ds(16 * t, 16)]
            cnt = plsc.load_gather(rowhist_v, [code], mask=m)
            lv = plsc.load_gather(lut_v, [cnt], mask=m)
            if m is not None:
                lv = jnp.where(m, lv, 0.0)
            acc = acc + lv
        for k in range(RH // 16):
            rowhist_v[pl.ds(16 * k, 16)] = zeros16i
        me_i = jnp.sum(acc)
        plsc.store_scatter(me_v, [jnp.full((16,), i, jnp.int32)],
                           jnp.full((16,), me_i, jnp.float32),
                           mask=iota16 == 0)
        return carry

    lax.fori_loop(0, rows_per, row_body, 0)

    pltpu.sync_copy(me_v, me_hbm.at[pl.ds(base, rows_per)])
    pltpu.sync_copy(tothist_v, gh_hbm.at[wid])


def _sc_call(ids):
    info = plsc.get_sparse_core_info()
    nc, ns = info.num_cores, info.num_subcores
    nw = nc * ns
    rows_per = B // nw
    mesh = plsc.VectorSubcoreMesh(core_axis_name="c", subcore_axis_name="s")
    fn = pl.kernel(
        functools.partial(_sc_body, rows_per, nc),
        out_type=[
            jax.ShapeDtypeStruct((B,), jnp.float32),
            jax.ShapeDtypeStruct((nw, GH), jnp.float32),
        ],
        mesh=mesh,
        compiler_params=pltpu.CompilerParams(needs_layout_passes=False),
        scratch_types=[
            pltpu.VMEM((rows_per, S), jnp.int32),
            pltpu.VMEM((256,), jnp.float32),
            pltpu.VMEM((16 * NCHUNK,), jnp.int32),
            pltpu.VMEM((RH,), jnp.int32),
            pltpu.VMEM((GH,), jnp.float32),
            pltpu.VMEM((rows_per,), jnp.float32),
        ],
    )
    return fn(ids)


def _graph_body(nw, hists_ref, me_ref, embed_ref, w1_ref, asrc1_ref,
                adst1_ref, b1_ref, w2_ref, asrc2_ref, adst2_ref, b2_ref,
                wr1_ref, br1_ref, wr2_ref, br2_ref, we_ref, be_ref,
                wotop_ref, wobot_ref, bo_ref, out_ref):
    f32 = jnp.float32

    gh = hists_ref[0:1, :]
    for n in range(1, nw):
        gh = gh + hists_ref[n:n + 1, :]
    ghd = jnp.concatenate([gh[0:1, 32 * i:32 * i + A] for i in range(A)],
                          axis=0)
    ri = lax.broadcasted_iota(jnp.int32, (A, A), 0)
    ci = lax.broadcasted_iota(jnp.int32, (A, A), 1)
    maskt = (ghd > 0.0) | (ri == ci)

    def gat_attention(adst_col, asrc_row, hsrc):
        e = adst_col + asrc_row
        e = jnp.where(e >= 0.0, e, 0.2 * e)
        e = jnp.where(maskt, e, -1e9)
        m = jnp.max(e, axis=1, keepdims=True)
        p = jnp.exp(e - m)
        att = p / jnp.sum(p, axis=1, keepdims=True)
        return jnp.dot(att, hsrc, preferred_element_type=f32)

    dn_t = (((1,), (1,)), ((), ()))

    x = embed_ref[...]
    h1 = jnp.dot(x, w1_ref[...], preferred_element_type=f32)
    heads = []
    for h in range(4):
        hh = h1[:, 64 * h:64 * h + 64]
        asrc = lax.dot_general(asrc1_ref[h:h + 1, :], hh, dn_t,
                               preferred_element_type=f32)
        adst = lax.dot_general(hh, adst1_ref[h:h + 1, :], dn_t,
                               preferred_element_type=f32)
        heads.append(gat_attention(adst, asrc, hh))
    h1o = jnp.concatenate(heads, axis=1) + b1_ref[...]
    h1o = jnp.where(h1o > 0.0, h1o, jnp.exp(h1o) - 1.0)

    h2 = jnp.dot(h1o, w2_ref[...], preferred_element_type=f32)
    asrc2 = lax.dot_general(asrc2_ref[...], h2, dn_t,
                            preferred_element_type=f32)
    adst2 = lax.dot_general(h2, adst2_ref[...], dn_t,
                            preferred_element_type=f32)
    h2o = gat_attention(adst2, asrc2, h2) + b2_ref[...]

    hflat = jnp.concatenate([h2o[n:n + 1, :] for n in range(A)], axis=1)
    g1 = jnp.dot(hflat, wr1_ref[...], preferred_element_type=f32)
    g1 = g1 + br1_ref[...]
    g1 = jnp.maximum(g1, 0.0)
    g2 = jnp.dot(g1, wr2_ref[...], preferred_element_type=f32) + br2_ref[...]

    const = (jnp.dot(g2, wotop_ref[...], preferred_element_type=f32)
             + jnp.dot(be_ref[...], wobot_ref[...],
                       preferred_element_type=f32)
             + bo_ref[...])
    v = jnp.dot(we_ref[...], wobot_ref[...],
                preferred_element_type=f32)

    out_ref[...] = const + me_ref[...] * v


def _graph_call(hists3, me2, embed, w1, a_src1, a_dst1, b1r, w2, a_src2,
                a_dst2, b2r, wr1, br1r, wr2, br2r, we, ber, wotop, wobot,
                bor):
    nw = hists3.shape[0]
    return pl.pallas_call(
        functools.partial(_graph_body, nw),
        out_shape=jax.ShapeDtypeStruct((B, D), jnp.float32),
    )(hists3, me2, embed, w1, a_src1, a_dst1, b1r, w2, a_src2, a_dst2, b2r,
      wr1, br1r, wr2, br2r, we, ber, wotop, wobot, bor)


def kernel(activity_ids, embed, W1, a_src1, a_dst1, b1, W2, a_src2, a_dst2,
           b2, Wr1, br1, Wr2, br2, We, be, Wo, bo):
    me, hists = _sc_call(activity_ids)
    h_mp = _graph_call(
        hists, me.reshape(B, 1), embed, W1,
        a_src1, a_dst1, b1.reshape(1, 256), W2, a_src2, a_dst2,
        b2.reshape(1, D), Wr1, br1.reshape(1, 256), Wr2, br2.reshape(1, D),
        We, be.reshape(1, 32), Wo[:D], Wo[D:], bo.reshape(1, D))
    return h_mp, me


# speedup vs baseline: 55.6068x; 1.0528x over previous
import functools

import jax
import jax.numpy as jnp
from jax import lax
from jax.experimental import pallas as pl
from jax.experimental.pallas import tpu as pltpu
from jax.experimental.pallas import tpu_sc as plsc

A = 21
D = 128
B = 4096
S = 200
T = S - 1
GH = 1024
RH = 448
NCHUNK = (T + 15) // 16


def _sc_body(rows_per, nc, ids_hbm, me_hbm, me2_hbm, gh_hbm,
             ids_v, lut_v, codes_v, rowhist_v, tothist_v, me_v):
    cid = lax.axis_index("c")
    sid = lax.axis_index("s")
    wid = sid * nc + cid
    base = wid * rows_per

    pltpu.sync_copy(ids_hbm.at[pl.ds(base, rows_per)], ids_v)

    zeros16i = jnp.zeros((16,), jnp.int32)
    ones16i = jnp.ones((16,), jnp.int32)
    ones16f = jnp.ones((16,), jnp.float32)
    iota16 = lax.iota(jnp.int32, 16)

    for k in range(256 // 16):
        x = (k * 16 + iota16).astype(jnp.float32) * (1.0 / T) + 1e-10
        bits = plsc.bitcast(x, jnp.int32)
        ex = (bits >> 23) - 127
        mant = plsc.bitcast((bits & 0x7FFFFF) | 0x3F800000, jnp.float32)
        adj = mant >= 1.4142135
        mant = jnp.where(adj, 0.5 * mant, mant)
        ef = ex.astype(jnp.float32) + jnp.where(adj, 1.0, 0.0)
        t = (mant - 1.0) / (mant + 1.0)
        t2 = t * t
        lnm = t * (2.0 + t2 * (2.0 / 3.0 + t2 * (2.0 / 5.0 + t2 * (2.0 / 7.0))))
        lnx = ef * 0.69314718 + lnm
        lut_v[pl.ds(16 * k, 16)] = lnx * (-1.0 / T)

    for k in range(RH // 16):
        rowhist_v[pl.ds(16 * k, 16)] = zeros16i
    for k in range(GH // 16):
        tothist_v[pl.ds(16 * k, 16)] = jnp.zeros((16,), jnp.float32)

    mlast = iota16 < (T - 16 * (NCHUNK - 1))

    def row_body(i, carry):
        rowi = jnp.full((16,), i, jnp.int32)
        for t in range(NCHUNK):
            col = 16 * t + iota16
            if t == NCHUNK - 1:
                m = mlast
                ca = jnp.minimum(col, T - 1)
                cb = jnp.minimum(col + 1, T)
            else:
                m = None
                ca = col
                cb = col + 1
            a = plsc.load_gather(ids_v, [rowi, ca], mask=m)
            b = plsc.load_gather(ids_v, [rowi, cb], mask=m)
            code = a * A + b
            codes_v[pl.ds(16 * t, 16)] = code
            plsc.addupdate_scatter(rowhist_v, [code], ones16i, mask=m)
            gcode = (b << 5) + a
            plsc.addupdate_scatter(tothist_v, [gcode], ones16f, mask=m)
        acc = jnp.zeros((16,), jnp.float32)
        for t in range(NCHUNK):
            m = mlast if t == NCHUNK - 1 else None
            code = codes_v[pl.ds(16 * t, 16)]
            cnt = plsc.load_gather(rowhist_v, [code], mask=m)
            lv = plsc.load_gather(lut_v, [cnt], mask=m)
            if m is not None:
                lv = jnp.where(m, lv, 0.0)
            acc = acc + lv
        for k in range(RH // 16):
            rowhist_v[pl.ds(16 * k, 16)] = zeros16i
        me_i = jnp.sum(acc)
        plsc.store_scatter(me_v, [jnp.full((16,), i, jnp.int32)],
                           jnp.full((16,), me_i, jnp.float32),
                           mask=iota16 == 0)
        return carry

    lax.fori_loop(0, rows_per, row_body, 0)

    pltpu.sync_copy(me_v, me_hbm.at[pl.ds(base, rows_per)])
    pltpu.sync_copy(me_v, me2_hbm.at[wid])
    pltpu.sync_copy(tothist_v, gh_hbm.at[wid])


def _sc_call(ids):
    info = plsc.get_sparse_core_info()
    nc, ns = info.num_cores, info.num_subcores
    nw = nc * ns
    rows_per = B // nw
    mesh = plsc.VectorSubcoreMesh(core_axis_name="c", subcore_axis_name="s")
    fn = pl.kernel(
        functools.partial(_sc_body, rows_per, nc),
        out_type=[
            jax.ShapeDtypeStruct((B,), jnp.float32),
            jax.ShapeDtypeStruct((nw, rows_per), jnp.float32),
            jax.ShapeDtypeStruct((nw, GH), jnp.float32),
        ],
        mesh=mesh,
        compiler_params=pltpu.CompilerParams(needs_layout_passes=False),
        scratch_types=[
            pltpu.VMEM((rows_per, S), jnp.int32),
            pltpu.VMEM((256,), jnp.float32),
            pltpu.VMEM((16 * NCHUNK,), jnp.int32),
            pltpu.VMEM((RH,), jnp.int32),
            pltpu.VMEM((GH,), jnp.float32),
            pltpu.VMEM((rows_per,), jnp.float32),
        ],
    )
    return fn(ids)


def _graph_body(nw, hists_ref, me_ref, embed_ref, w1_ref, asrc1_ref,
                adst1_ref, b1_ref, w2_ref, asrc2_ref, adst2_ref, b2_ref,
                wr1_ref, br1_ref, wr2_ref, br2_ref, we_ref, be_ref,
                wotop_ref, wobot_ref, bo_ref, out_ref):
    f32 = jnp.float32

    gh = hists_ref[0:1, :]
    for n in range(1, nw):
        gh = gh + hists_ref[n:n + 1, :]
    ghd = jnp.concatenate([gh[0:1, 32 * i:32 * i + A] for i in range(A)],
                          axis=0)
    ri = lax.broadcasted_iota(jnp.int32, (A, A), 0)
    ci = lax.broadcasted_iota(jnp.int32, (A, A), 1)
    maskt = (ghd > 0.0) | (ri == ci)

    def gat_attention(adst_col, asrc_row, hsrc):
        e = adst_col + asrc_row
        e = jnp.where(e >= 0.0, e, 0.2 * e)
        e = jnp.where(maskt, e, -1e9)
        m = jnp.max(e, axis=1, keepdims=True)
        p = jnp.exp(e - m)
        att = p / jnp.sum(p, axis=1, keepdims=True)
        return jnp.dot(att, hsrc, preferred_element_type=f32)

    dn_t = (((1,), (1,)), ((), ()))

    x = embed_ref[...]
    h1 = jnp.dot(x, w1_ref[...], preferred_element_type=f32)
    heads = []
    for h in range(4):
        hh = h1[:, 64 * h:64 * h + 64]
        asrc = lax.dot_general(asrc1_ref[h:h + 1, :], hh, dn_t,
                               preferred_element_type=f32)
        adst = lax.dot_general(hh, adst1_ref[h:h + 1, :], dn_t,
                               preferred_element_type=f32)
        heads.append(gat_attention(adst, asrc, hh))
    h1o = jnp.concatenate(heads, axis=1) + b1_ref[...]
    h1o = jnp.where(h1o > 0.0, h1o, jnp.exp(h1o) - 1.0)

    h2 = jnp.dot(h1o, w2_ref[...], preferred_element_type=f32)
    asrc2 = lax.dot_general(asrc2_ref[...], h2, dn_t,
                            preferred_element_type=f32)
    adst2 = lax.dot_general(h2, adst2_ref[...], dn_t,
                            preferred_element_type=f32)
    h2o = gat_attention(adst2, asrc2, h2) + b2_ref[...]

    hflat = jnp.concatenate([h2o[n:n + 1, :] for n in range(A)], axis=1)
    g1 = jnp.dot(hflat, wr1_ref[...], preferred_element_type=f32)
    g1 = g1 + br1_ref[...]
    g1 = jnp.maximum(g1, 0.0)
    g2 = jnp.dot(g1, wr2_ref[...], preferred_element_type=f32) + br2_ref[...]

    const = (jnp.dot(g2, wotop_ref[...], preferred_element_type=f32)
             + jnp.dot(be_ref[...], wobot_ref[...],
                       preferred_element_type=f32)
             + bo_ref[...])
    v = jnp.dot(we_ref[...], wobot_ref[...],
                preferred_element_type=f32)

    dn_outer = (((0,), (0,)), ((), ()))
    nblk = me_ref.shape[0]
    rows = me_ref.shape[1]
    for r in range(nblk):
        mev = lax.dot_general(me_ref[r:r + 1, :], v, dn_outer,
                              preferred_element_type=f32)
        out_ref[pl.ds(rows * r, rows), :] = const + mev


def _graph_call(hists3, me2, embed, w1, a_src1, a_dst1, b1r, w2, a_src2,
                a_dst2, b2r, wr1, br1r, wr2, br2r, we, ber, wotop, wobot,
                bor):
    nw = hists3.shape[0]
    return pl.pallas_call(
        functools.partial(_graph_body, nw),
        out_shape=jax.ShapeDtypeStruct((B, D), jnp.float32),
    )(hists3, me2, embed, w1, a_src1, a_dst1, b1r, w2, a_src2, a_dst2, b2r,
      wr1, br1r, wr2, br2r, we, ber, wotop, wobot, bor)


def kernel(activity_ids, embed, W1, a_src1, a_dst1, b1, W2, a_src2, a_dst2,
           b2, Wr1, br1, Wr2, br2, We, be, Wo, bo):
    me, me2, hists = _sc_call(activity_ids)
    h_mp = _graph_call(
        hists, me2, embed, W1,
        a_src1, a_dst1, b1.reshape(1, 256), W2, a_src2, a_dst2,
        b2.reshape(1, D), Wr1, br1.reshape(1, 256), Wr2, br2.reshape(1, D),
        We, be.reshape(1, 32), Wo[:D], Wo[D:], bo.reshape(1, D))
    return h_mp, me


# speedup vs baseline: 56.7710x; 1.0209x over previous
import functools

import jax
import jax.numpy as jnp
from jax import lax
from jax.experimental import pallas as pl
from jax.experimental.pallas import tpu as pltpu
from jax.experimental.pallas import tpu_sc as plsc

A = 21
D = 128
B = 4096
S = 200
T = S - 1
GH = 448
RH = 448
SP = 216
NCHUNK = (T + 15) // 16


def _sc_body(rows_per, nc, ids_hbm, me_hbm, me2_hbm, gh_hbm,
             ids_v, lut_v, codes_v, rowhist_v, tothist_v, me_v):
    cid = lax.axis_index("c")
    sid = lax.axis_index("s")
    wid = sid * nc + cid
    base = wid * rows_per

    pltpu.sync_copy(ids_hbm.at[pl.ds(base, rows_per)], ids_v)

    zeros16i = jnp.zeros((16,), jnp.int32)
    ones16i = jnp.ones((16,), jnp.int32)
    ones16f = jnp.ones((16,), jnp.float32)
    iota16 = lax.iota(jnp.int32, 16)

    for k in range(256 // 16):
        x = (k * 16 + iota16).astype(jnp.float32) * (1.0 / T) + 1e-10
        bits = plsc.bitcast(x, jnp.int32)
        ex = (bits >> 23) - 127
        mant = plsc.bitcast((bits & 0x7FFFFF) | 0x3F800000, jnp.float32)
        adj = mant >= 1.4142135
        mant = jnp.where(adj, 0.5 * mant, mant)
        ef = ex.astype(jnp.float32) + jnp.where(adj, 1.0, 0.0)
        t = (mant - 1.0) / (mant + 1.0)
        t2 = t * t
        lnm = t * (2.0 + t2 * (2.0 / 3.0 + t2 * (2.0 / 5.0 + t2 * (2.0 / 7.0))))
        lnx = ef * 0.69314718 + lnm
        lut_v[pl.ds(16 * k, 16)] = lnx * (-1.0 / T)

    for k in range(RH // 16):
        rowhist_v[pl.ds(16 * k, 16)] = zeros16i
    for k in range(GH // 16):
        tothist_v[pl.ds(16 * k, 16)] = jnp.zeros((16,), jnp.float32)

    mlast = iota16 >= (16 * NCHUNK - T)

    def row_body(i, carry):
        for t in range(NCHUNK):
            off = 16 * t if t < NCHUNK - 1 else T - 16
            m = mlast if t == NCHUNK - 1 else None
            a = ids_v[i, pl.ds(off, 16)]
            b = ids_v[i, pl.ds(off + 1, 16)]
            code = a * A + b
            codes_v[pl.ds(16 * t, 16)] = code
            plsc.addupdate_scatter(rowhist_v, [code], ones16i, mask=m)
            plsc.addupdate_scatter(tothist_v, [code], ones16f, mask=m)
        acc = jnp.zeros((16,), jnp.float32)
        for t in range(NCHUNK):
            m = mlast if t == NCHUNK - 1 else None
            code = codes_v[pl.ds(16 * t, 16)]
            cnt = plsc.load_gather(rowhist_v, [code], mask=m)
            lv = plsc.load_gather(lut_v, [cnt], mask=m)
            if m is not None:
                lv = jnp.where(m, lv, 0.0)
            acc = acc + lv
        for k in range(RH // 16):
            rowhist_v[pl.ds(16 * k, 16)] = zeros16i
        me_i = jnp.sum(acc)
        plsc.store_scatter(me_v, [jnp.full((16,), i, jnp.int32)],
                           jnp.full((16,), me_i, jnp.float32),
                           mask=iota16 == 0)
        return carry

    lax.fori_loop(0, rows_per, row_body, 0)

    pltpu.sync_copy(me_v, me_hbm.at[pl.ds(base, rows_per)])
    pltpu.sync_copy(me_v, me2_hbm.at[wid])
    pltpu.sync_copy(tothist_v, gh_hbm.at[wid])


def _sc_call(ids):
    info = plsc.get_sparse_core_info()
    nc, ns = info.num_cores, info.num_subcores
    nw = nc * ns
    rows_per = B // nw
    mesh = plsc.VectorSubcoreMesh(core_axis_name="c", subcore_axis_name="s")
    fn = pl.kernel(
        functools.partial(_sc_body, rows_per, nc),
        out_type=[
            jax.ShapeDtypeStruct((B,), jnp.float32),
            jax.ShapeDtypeStruct((nw, rows_per), jnp.float32),
            jax.ShapeDtypeStruct((nw, GH), jnp.float32),
        ],
        mesh=mesh,
        compiler_params=pltpu.CompilerParams(needs_layout_passes=False),
        scratch_types=[
            pltpu.VMEM((rows_per, S), jnp.int32),
            pltpu.VMEM((256,), jnp.float32),
            pltpu.VMEM((16 * NCHUNK,), jnp.int32),
            pltpu.VMEM((RH,), jnp.int32),
            pltpu.VMEM((GH,), jnp.float32),
            pltpu.VMEM((rows_per,), jnp.float32),
        ],
    )
    return fn(ids)


def _graph_body(nw, hists_ref, me_ref, embed_ref, w1_ref, asrc1_ref,
                adst1_ref, b1_ref, w2_ref, asrc2_ref, adst2_ref, b2_ref,
                wr1_ref, br1_ref, wr2_ref, br2_ref, we_ref, be_ref,
                wotop_ref, wobot_ref, bo_ref, out_ref):
    f32 = jnp.float32

    gh = hists_ref[0:1, :]
    for n in range(1, nw):
        gh = gh + hists_ref[n:n + 1, :]
    ghm = jnp.concatenate([gh[0:1, A * s:A * s + A] for s in range(A)],
                          axis=0)
    ri = lax.broadcasted_iota(jnp.int32, (A, A), 0)
    ci = lax.broadcasted_iota(jnp.int32, (A, A), 1)
    mask = (ghm > 0.0) | (ri == ci)

    def gat_attention(asrc_col, adst_row, hsrc):
        f = asrc_col + adst_row
        f = jnp.where(f >= 0.0, f, 0.2 * f)
        f = jnp.where(mask, f, -1e9)
        m = jnp.max(f, axis=0, keepdims=True)
        p = jnp.exp(f - m)
        att = p / jnp.sum(p, axis=0, keepdims=True)
        return lax.dot_general(att, hsrc, (((0,), (0,)), ((), ())),
                               preferred_element_type=f32)

    dn_t = (((1,), (1,)), ((), ()))

    x = embed_ref[...]
    h1 = jnp.dot(x, w1_ref[...], preferred_element_type=f32)
    heads = []
    for h in range(4):
        hh = h1[:, 64 * h:64 * h + 64]
        asrc = lax.dot_general(hh, asrc1_ref[h:h + 1, :], dn_t,
                               preferred_element_type=f32)
        adst = lax.dot_general(adst1_ref[h:h + 1, :], hh, dn_t,
                               preferred_element_type=f32)
        heads.append(gat_attention(asrc, adst, hh))
    h1o = jnp.concatenate(heads, axis=1) + b1_ref[...]
    h1o = jnp.where(h1o > 0.0, h1o, jnp.exp(h1o) - 1.0)

    h2 = jnp.dot(h1o, w2_ref[...], preferred_element_type=f32)
    asrc2 = lax.dot_general(h2, asrc2_ref[...], dn_t,
                            preferred_element_type=f32)
    adst2 = lax.dot_general(adst2_ref[...], h2, dn_t,
                            preferred_element_type=f32)
    h2o = gat_attention(asrc2, adst2, h2) + b2_ref[...]

    hflat = jnp.concatenate([h2o[n:n + 1, :] for n in range(A)], axis=1)
    g1 = jnp.dot(hflat, wr1_ref[...], preferred_element_type=f32)
    g1 = g1 + br1_ref[...]
    g1 = jnp.maximum(g1, 0.0)
    g2 = jnp.dot(g1, wr2_ref[...], preferred_element_type=f32) + br2_ref[...]

    const = (jnp.dot(g2, wotop_ref[...], preferred_element_type=f32)
             + jnp.dot(be_ref[...], wobot_ref[...],
                       preferred_element_type=f32)
             + bo_ref[...])
    v = jnp.dot(we_ref[...], wobot_ref[...],
                preferred_element_type=f32)

    dn_outer = (((0,), (0,)), ((), ()))
    nblk = me_ref.shape[0]
    rows = me_ref.shape[1]
    for r in range(nblk):
        mev = lax.dot_general(me_ref[r:r + 1, :], v, dn_outer,
                              preferred_element_type=f32)
        out_ref[pl.ds(rows * r, rows), :] = const + mev


def _graph_call(hists3, me2, embed, w1, a_src1, a_dst1, b1r, w2, a_src2,
                a_dst2, b2r, wr1, br1r, wr2, br2r, we, ber, wotop, wobot,
                bor):
    nw = hists3.shape[0]
    return pl.pallas_call(
        functools.partial(_graph_body, nw),
        out_shape=jax.ShapeDtypeStruct((B, D), jnp.float32),
    )(hists3, me2, embed, w1, a_src1, a_dst1, b1r, w2, a_src2, a_dst2, b2r,
      wr1, br1r, wr2, br2r, we, ber, wotop, wobot, bor)


def kernel(activity_ids, embed, W1, a_src1, a_dst1, b1, W2, a_src2, a_dst2,
           b2, Wr1, br1, Wr2, br2, We, be, Wo, bo):
    me, me2, hists = _sc_call(activity_ids)
    h_mp = _graph_call(
        hists, me2, embed, W1,
        a_src1, a_dst1, b1.reshape(1, 256), W2, a_src2, a_dst2,
        b2.reshape(1, D), Wr1, br1.reshape(1, 256), Wr2, br2.reshape(1, D),
        We, be.reshape(1, 32), Wo[:D], Wo[D:], bo.reshape(1, D))
    return h_mp, me


# speedup vs baseline: 63.2125x; 1.1135x over previous
import functools

import jax
import jax.numpy as jnp
from jax import lax
from jax.experimental import pallas as pl
from jax.experimental.pallas import tpu as pltpu
from jax.experimental.pallas import tpu_sc as plsc

A = 21
D = 128
B = 4096
S = 200
T = S - 1
GH = 448
RH = 448
SP = 216
NCHUNK = (T + 15) // 16


def _sc_body(rows_per, nc, ids_hbm, me_hbm, me2_hbm, gh_hbm,
             ids_v, lut_v, codes_v, rowhist_v, tothist_v, me_v):
    cid = lax.axis_index("c")
    sid = lax.axis_index("s")
    wid = sid * nc + cid
    base = wid * rows_per

    pltpu.sync_copy(ids_hbm.at[pl.ds(base, rows_per)], ids_v)

    zeros16i = jnp.zeros((16,), jnp.int32)
    ones16i = jnp.ones((16,), jnp.int32)
    ones16f = jnp.ones((16,), jnp.float32)
    iota16 = lax.iota(jnp.int32, 16)

    for k in range(256 // 16):
        x = (k * 16 + iota16).astype(jnp.float32) * (1.0 / T) + 1e-10
        bits = plsc.bitcast(x, jnp.int32)
        ex = (bits >> 23) - 127
        mant = plsc.bitcast((bits & 0x7FFFFF) | 0x3F800000, jnp.float32)
        adj = mant >= 1.4142135
        mant = jnp.where(adj, 0.5 * mant, mant)
        ef = ex.astype(jnp.float32) + jnp.where(adj, 1.0, 0.0)
        t = (mant - 1.0) / (mant + 1.0)
        t2 = t * t
        lnm = t * (2.0 + t2 * (2.0 / 3.0 + t2 * (2.0 / 5.0 + t2 * (2.0 / 7.0))))
        lnx = ef * 0.69314718 + lnm
        lut_v[pl.ds(16 * k, 16)] = lnx * (-1.0 / T)

    for k in range(2 * RH // 16):
        rowhist_v[pl.ds(16 * k, 16)] = zeros16i
    for k in range(GH // 16):
        tothist_v[pl.ds(16 * k, 16)] = jnp.zeros((16,), jnp.float32)

    mlast = iota16 >= (16 * NCHUNK - T)

    def pair_body(p, carry):
        i0 = 2 * p
        i1 = 2 * p + 1
        for t in range(NCHUNK):
            off = 16 * t if t < NCHUNK - 1 else T - 16
            m = mlast if t == NCHUNK - 1 else None
            a0 = ids_v[i0, pl.ds(off, 16)]
            b0 = ids_v[i0, pl.ds(off + 1, 16)]
            a1 = ids_v[i1, pl.ds(off, 16)]
            b1 = ids_v[i1, pl.ds(off + 1, 16)]
            code0 = a0 * A + b0
            code1 = a1 * A + b1 + RH
            codes_v[pl.ds(16 * t, 16)] = code0
            codes_v[pl.ds(16 * NCHUNK + 16 * t, 16)] = code1
            plsc.addupdate_scatter(rowhist_v, [code0], ones16i, mask=m)
            plsc.addupdate_scatter(rowhist_v, [code1], ones16i, mask=m)
            plsc.addupdate_scatter(tothist_v, [code0], ones16f, mask=m)
            plsc.addupdate_scatter(tothist_v, [code1 - RH], ones16f, mask=m)
        acc0 = jnp.zeros((16,), jnp.float32)
        acc1 = jnp.zeros((16,), jnp.float32)
        for t in range(NCHUNK):
            m = mlast if t == NCHUNK - 1 else None
            code0 = codes_v[pl.ds(16 * t, 16)]
            code1 = codes_v[pl.ds(16 * NCHUNK + 16 * t, 16)]
            cnt0 = plsc.load_gather(rowhist_v, [code0], mask=m)
            cnt1 = plsc.load_gather(rowhist_v, [code1], mask=m)
            lv0 = plsc.load_gather(lut_v, [cnt0], mask=m)
            lv1 = plsc.load_gather(lut_v, [cnt1], mask=m)
            if m is not None:
                lv0 = jnp.where(m, lv0, 0.0)
                lv1 = jnp.where(m, lv1, 0.0)
            acc0 = acc0 + lv0
            acc1 = acc1 + lv1
        for k in range(2 * RH // 16):
            rowhist_v[pl.ds(16 * k, 16)] = zeros16i
        me0 = jnp.sum(acc0)
        me1 = jnp.sum(acc1)
        plsc.store_scatter(me_v, [jnp.full((16,), i0, jnp.int32)],
                           jnp.full((16,), me0, jnp.float32),
                           mask=iota16 == 0)
        plsc.store_scatter(me_v, [jnp.full((16,), i1, jnp.int32)],
                           jnp.full((16,), me1, jnp.float32),
                           mask=iota16 == 0)
        return carry

    lax.fori_loop(0, rows_per // 2, pair_body, 0)

    pltpu.sync_copy(me_v, me_hbm.at[pl.ds(base, rows_per)])
    pltpu.sync_copy(me_v, me2_hbm.at[wid])
    pltpu.sync_copy(tothist_v, gh_hbm.at[wid])


def _sc_call(ids):
    info = plsc.get_sparse_core_info()
    nc, ns = info.num_cores, info.num_subcores
    nw = nc * ns
    rows_per = B // nw
    mesh = plsc.VectorSubcoreMesh(core_axis_name="c", subcore_axis_name="s")
    fn = pl.kernel(
        functools.partial(_sc_body, rows_per, nc),
        out_type=[
            jax.ShapeDtypeStruct((B,), jnp.float32),
            jax.ShapeDtypeStruct((nw, rows_per), jnp.float32),
            jax.ShapeDtypeStruct((nw, GH), jnp.float32),
        ],
        mesh=mesh,
        compiler_params=pltpu.CompilerParams(needs_layout_passes=False),
        scratch_types=[
            pltpu.VMEM((rows_per, S), jnp.int32),
            pltpu.VMEM((256,), jnp.float32),
            pltpu.VMEM((2 * 16 * NCHUNK,), jnp.int32),
            pltpu.VMEM((2 * RH,), jnp.int32),
            pltpu.VMEM((GH,), jnp.float32),
            pltpu.VMEM((rows_per,), jnp.float32),
        ],
    )
    return fn(ids)


def _graph_body(nw, hists_ref, me_ref, embed_ref, w1_ref, asrc1_ref,
                adst1_ref, b1_ref, w2_ref, asrc2_ref, adst2_ref, b2_ref,
                wr1_ref, br1_ref, wr2_ref, br2_ref, we_ref, be_ref,
                wotop_ref, wobot_ref, bo_ref, out_ref):
    f32 = jnp.float32

    gh = hists_ref[0:1, :]
    for n in range(1, nw):
        gh = gh + hists_ref[n:n + 1, :]
    ghm = jnp.concatenate([gh[0:1, A * s:A * s + A] for s in range(A)],
                          axis=0)
    ri = lax.broadcasted_iota(jnp.int32, (A, A), 0)
    ci = lax.broadcasted_iota(jnp.int32, (A, A), 1)
    mask = (ghm > 0.0) | (ri == ci)

    def gat_attention(asrc_col, adst_row, hsrc):
        f = asrc_col + adst_row
        f = jnp.where(f >= 0.0, f, 0.2 * f)
        f = jnp.where(mask, f, -1e9)
        m = jnp.max(f, axis=0, keepdims=True)
        p = jnp.exp(f - m)
        att = p / jnp.sum(p, axis=0, keepdims=True)
        return lax.dot_general(att, hsrc, (((0,), (0,)), ((), ())),
                               preferred_element_type=f32)

    dn_t = (((1,), (1,)), ((), ()))

    x = embed_ref[...]
    h1 = jnp.dot(x, w1_ref[...], preferred_element_type=f32)
    heads = []
    for h in range(4):
        hh = h1[:, 64 * h:64 * h + 64]
        asrc = lax.dot_general(hh, asrc1_ref[h:h + 1, :], dn_t,
                               preferred_element_type=f32)
        adst = lax.dot_general(adst1_ref[h:h + 1, :], hh, dn_t,
                               preferred_element_type=f32)
        heads.append(gat_attention(asrc, adst, hh))
    h1o = jnp.concatenate(heads, axis=1) + b1_ref[...]
    h1o = jnp.where(h1o > 0.0, h1o, jnp.exp(h1o) - 1.0)

    h2 = jnp.dot(h1o, w2_ref[...], preferred_element_type=f32)
    asrc2 = lax.dot_general(h2, asrc2_ref[...], dn_t,
                            preferred_element_type=f32)
    adst2 = lax.dot_general(adst2_ref[...], h2, dn_t,
                            preferred_element_type=f32)
    h2o = gat_attention(asrc2, adst2, h2) + b2_ref[...]

    hflat = jnp.concatenate([h2o[n:n + 1, :] for n in range(A)], axis=1)
    g1 = jnp.dot(hflat, wr1_ref[...], preferred_element_type=f32)
    g1 = g1 + br1_ref[...]
    g1 = jnp.maximum(g1, 0.0)
    g2 = jnp.dot(g1, wr2_ref[...], preferred_element_type=f32) + br2_ref[...]

    const = (jnp.dot(g2, wotop_ref[...], preferred_element_type=f32)
             + jnp.dot(be_ref[...], wobot_ref[...],
                       preferred_element_type=f32)
             + bo_ref[...])
    v = jnp.dot(we_ref[...], wobot_ref[...],
                preferred_element_type=f32)

    dn_outer = (((0,), (0,)), ((), ()))
    nblk = me_ref.shape[0]
    rows = me_ref.shape[1]
    for r in range(nblk):
        mev = lax.dot_general(me_ref[r:r + 1, :], v, dn_outer,
                              preferred_element_type=f32)
        out_ref[pl.ds(rows * r, rows), :] = const + mev


def _graph_call(hists3, me2, embed, w1, a_src1, a_dst1, b1r, w2, a_src2,
                a_dst2, b2r, wr1, br1r, wr2, br2r, we, ber, wotop, wobot,
                bor):
    nw = hists3.shape[0]
    return pl.pallas_call(
        functools.partial(_graph_body, nw),
        out_shape=jax.ShapeDtypeStruct((B, D), jnp.float32),
    )(hists3, me2, embed, w1, a_src1, a_dst1, b1r, w2, a_src2, a_dst2, b2r,
      wr1, br1r, wr2, br2r, we, ber, wotop, wobot, bor)


def kernel(activity_ids, embed, W1, a_src1, a_dst1, b1, W2, a_src2, a_dst2,
           b2, Wr1, br1, Wr2, br2, We, be, Wo, bo):
    me, me2, hists = _sc_call(activity_ids)
    h_mp = _graph_call(
        hists, me2, embed, W1,
        a_src1, a_dst1, b1.reshape(1, 256), W2, a_src2, a_dst2,
        b2.reshape(1, D), Wr1, br1.reshape(1, 256), Wr2, br2.reshape(1, D),
        We, be.reshape(1, 32), Wo[:D], Wo[D:], bo.reshape(1, D))
    return h_mp, me


# speedup vs baseline: 67.0963x; 1.0614x over previous
import functools

import jax
import jax.numpy as jnp
from jax import lax
from jax.experimental import pallas as pl
from jax.experimental.pallas import tpu as pltpu
from jax.experimental.pallas import tpu_sc as plsc

A = 21
D = 128
B = 4096
S = 200
T = S - 1
GH = 448
RH = 448
SP = 216
NCHUNK = (T + 15) // 16
NWAY = 4


def _sc_body(rows_per, nc, ids_hbm, me_hbm, me2_hbm, gh_hbm,
             ids_v, lut_v, codes_v, rowhist_v, tothist_v, me_v):
    cid = lax.axis_index("c")
    sid = lax.axis_index("s")
    wid = sid * nc + cid
    base = wid * rows_per

    pltpu.sync_copy(ids_hbm.at[pl.ds(base, rows_per)], ids_v)

    zeros16i = jnp.zeros((16,), jnp.int32)
    ones16i = jnp.ones((16,), jnp.int32)
    ones16f = jnp.ones((16,), jnp.float32)
    iota16 = lax.iota(jnp.int32, 16)

    for k in range(256 // 16):
        x = (k * 16 + iota16).astype(jnp.float32) * (1.0 / T) + 1e-10
        bits = plsc.bitcast(x, jnp.int32)
        ex = (bits >> 23) - 127
        mant = plsc.bitcast((bits & 0x7FFFFF) | 0x3F800000, jnp.float32)
        adj = mant >= 1.4142135
        mant = jnp.where(adj, 0.5 * mant, mant)
        ef = ex.astype(jnp.float32) + jnp.where(adj, 1.0, 0.0)
        t = (mant - 1.0) / (mant + 1.0)
        t2 = t * t
        lnm = t * (2.0 + t2 * (2.0 / 3.0 + t2 * (2.0 / 5.0 + t2 * (2.0 / 7.0))))
        lnx = ef * 0.69314718 + lnm
        lut_v[pl.ds(16 * k, 16)] = lnx * (-1.0 / T)

    for k in range(NWAY * RH // 16):
        rowhist_v[pl.ds(16 * k, 16)] = zeros16i
    for k in range(GH // 16):
        tothist_v[pl.ds(16 * k, 16)] = jnp.zeros((16,), jnp.float32)

    mlast = iota16 >= (16 * NCHUNK - T)

    def group_body(p, carry):
        iis = [NWAY * p + w for w in range(NWAY)]
        for t in range(NCHUNK):
            off = 16 * t if t < NCHUNK - 1 else T - 16
            m = mlast if t == NCHUNK - 1 else None
            codes = []
            for w, i in enumerate(iis):
                a = ids_v[i, pl.ds(off, 16)]
                b = ids_v[i, pl.ds(off + 1, 16)]
                codes.append(a * A + b + w * RH)
            for w in range(NWAY):
                codes_v[pl.ds(16 * (w * NCHUNK + t), 16)] = codes[w]
            for w in range(NWAY):
                plsc.addupdate_scatter(rowhist_v, [codes[w]], ones16i,
                                       mask=m)
            for w in range(NWAY):
                plsc.addupdate_scatter(tothist_v, [codes[w] - w * RH],
                                       ones16f, mask=m)
        accs = [jnp.zeros((16,), jnp.float32) for _ in range(NWAY)]
        for t in range(NCHUNK):
            m = mlast if t == NCHUNK - 1 else None
            codes = [codes_v[pl.ds(16 * (w * NCHUNK + t), 16)]
                     for w in range(NWAY)]
            cnts = [plsc.load_gather(rowhist_v, [codes[w]], mask=m)
                    for w in range(NWAY)]
            lvs = [plsc.load_gather(lut_v, [cnts[w]], mask=m)
                   for w in range(NWAY)]
            if m is not None:
                lvs = [jnp.where(m, lv, 0.0) for lv in lvs]
            accs = [acc + lv for acc, lv in zip(accs, lvs)]
        for k in range(NWAY * RH // 16):
            rowhist_v[pl.ds(16 * k, 16)] = zeros16i
        for w, i in enumerate(iis):
            plsc.store_scatter(me_v, [jnp.full((16,), i, jnp.int32)],
                               jnp.full((16,), jnp.sum(accs[w]), jnp.float32),
                               mask=iota16 == 0)
        return carry

    lax.fori_loop(0, rows_per // NWAY, group_body, 0)

    pltpu.sync_copy(me_v, me_hbm.at[pl.ds(base, rows_per)])
    pltpu.sync_copy(me_v, me2_hbm.at[wid])
    pltpu.sync_copy(tothist_v, gh_hbm.at[wid])


def _sc_call(ids):
    info = plsc.get_sparse_core_info()
    nc, ns = info.num_cores, info.num_subcores
    nw = nc * ns
    rows_per = B // nw
    mesh = plsc.VectorSubcoreMesh(core_axis_name="c", subcore_axis_name="s")
    fn = pl.kernel(
        functools.partial(_sc_body, rows_per, nc),
        out_type=[
            jax.ShapeDtypeStruct((B,), jnp.float32),
            jax.ShapeDtypeStruct((nw, rows_per), jnp.float32),
            jax.ShapeDtypeStruct((nw, GH), jnp.float32),
        ],
        mesh=mesh,
        compiler_params=pltpu.CompilerParams(needs_layout_passes=False),
        scratch_types=[
            pltpu.VMEM((rows_per, S), jnp.int32),
            pltpu.VMEM((256,), jnp.float32),
            pltpu.VMEM((NWAY * 16 * NCHUNK,), jnp.int32),
            pltpu.VMEM((NWAY * RH,), jnp.int32),
            pltpu.VMEM((GH,), jnp.float32),
            pltpu.VMEM((rows_per,), jnp.float32),
        ],
    )
    return fn(ids)


def _graph_body(nw, hists_ref, me_ref, embed_ref, w1_ref, asrc1_ref,
                adst1_ref, b1_ref, w2_ref, asrc2_ref, adst2_ref, b2_ref,
                wr1_ref, br1_ref, wr2_ref, br2_ref, we_ref, be_ref,
                wotop_ref, wobot_ref, bo_ref, out_ref):
    f32 = jnp.float32

    gh = hists_ref[0:1, :]
    for n in range(1, nw):
        gh = gh + hists_ref[n:n + 1, :]
    ghm = jnp.concatenate([gh[0:1, A * s:A * s + A] for s in range(A)],
                          axis=0)
    ri = lax.broadcasted_iota(jnp.int32, (A, A), 0)
    ci = lax.broadcasted_iota(jnp.int32, (A, A), 1)
    mask = (ghm > 0.0) | (ri == ci)

    def gat_attention(asrc_col, adst_row, hsrc):
        f = asrc_col + adst_row
        f = jnp.where(f >= 0.0, f, 0.2 * f)
        f = jnp.where(mask, f, -1e9)
        m = jnp.max(f, axis=0, keepdims=True)
        p = jnp.exp(f - m)
        att = p / jnp.sum(p, axis=0, keepdims=True)
        return lax.dot_general(att, hsrc, (((0,), (0,)), ((), ())),
                               preferred_element_type=f32)

    dn_t = (((1,), (1,)), ((), ()))

    x = embed_ref[...]
    h1 = jnp.dot(x, w1_ref[...], preferred_element_type=f32)
    heads = []
    for h in range(4):
        hh = h1[:, 64 * h:64 * h + 64]
        asrc = lax.dot_general(hh, asrc1_ref[h:h + 1, :], dn_t,
                               preferred_element_type=f32)
        adst = lax.dot_general(adst1_ref[h:h + 1, :], hh, dn_t,
                               preferred_element_type=f32)
        heads.append(gat_attention(asrc, adst, hh))
    h1o = jnp.concatenate(heads, axis=1) + b1_ref[...]
    h1o = jnp.where(h1o > 0.0, h1o, jnp.exp(h1o) - 1.0)

    h2 = jnp.dot(h1o, w2_ref[...], preferred_element_type=f32)
    asrc2 = lax.dot_general(h2, asrc2_ref[...], dn_t,
                            preferred_element_type=f32)
    adst2 = lax.dot_general(adst2_ref[...], h2, dn_t,
                            preferred_element_type=f32)
    h2o = gat_attention(asrc2, adst2, h2) + b2_ref[...]

    hflat = jnp.concatenate([h2o[n:n + 1, :] for n in range(A)], axis=1)
    g1 = jnp.dot(hflat, wr1_ref[...], preferred_element_type=f32)
    g1 = g1 + br1_ref[...]
    g1 = jnp.maximum(g1, 0.0)
    g2 = jnp.dot(g1, wr2_ref[...], preferred_element_type=f32) + br2_ref[...]

    const = (jnp.dot(g2, wotop_ref[...], preferred_element_type=f32)
             + jnp.dot(be_ref[...], wobot_ref[...],
                       preferred_element_type=f32)
             + bo_ref[...])
    v = jnp.dot(we_ref[...], wobot_ref[...],
                preferred_element_type=f32)

    dn_outer = (((0,), (0,)), ((), ()))
    nblk = me_ref.shape[0]
    rows = me_ref.shape[1]
    for r in range(nblk):
        mev = lax.dot_general(me_ref[r:r + 1, :], v, dn_outer,
                              preferred_element_type=f32)
        out_ref[pl.ds(rows * r, rows), :] = const + mev


def _graph_call(hists3, me2, embed, w1, a_src1, a_dst1, b1r, w2, a_src2,
                a_dst2, b2r, wr1, br1r, wr2, br2r, we, ber, wotop, wobot,
                bor):
    nw = hists3.shape[0]
    return pl.pallas_call(
        functools.partial(_graph_body, nw),
        out_shape=jax.ShapeDtypeStruct((B, D), jnp.float32),
    )(hists3, me2, embed, w1, a_src1, a_dst1, b1r, w2, a_src2, a_dst2, b2r,
      wr1, br1r, wr2, br2r, we, ber, wotop, wobot, bor)


def kernel(activity_ids, embed, W1, a_src1, a_dst1, b1, W2, a_src2, a_dst2,
           b2, Wr1, br1, Wr2, br2, We, be, Wo, bo):
    me, me2, hists = _sc_call(activity_ids)
    h_mp = _graph_call(
        hists, me2, embed, W1,
        a_src1, a_dst1, b1.reshape(1, 256), W2, a_src2, a_dst2,
        b2.reshape(1, D), Wr1, br1.reshape(1, 256), Wr2, br2.reshape(1, D),
        We, be.reshape(1, 32), Wo[:D], Wo[D:], bo.reshape(1, D))
    return h_mp, me


# speedup vs baseline: 67.1352x; 1.0006x over previous
import functools

import jax
import jax.numpy as jnp
from jax import lax
from jax.experimental import pallas as pl
from jax.experimental.pallas import tpu as pltpu
from jax.experimental.pallas import tpu_sc as plsc

A = 21
D = 128
B = 4096
S = 200
T = S - 1
GH = 448
RH = 448
SP = 216
NCHUNK = (T + 15) // 16
NWAY = 8


def _sc_body(rows_per, nc, ids_hbm, me_hbm, me2_hbm, gh_hbm,
             ids_v, lut_v, codes_v, rowhist_v, tothist_v, me_v):
    cid = lax.axis_index("c")
    sid = lax.axis_index("s")
    wid = sid * nc + cid
    base = wid * rows_per

    pltpu.sync_copy(ids_hbm.at[pl.ds(base, rows_per)], ids_v)

    zeros16i = jnp.zeros((16,), jnp.int32)
    ones16i = jnp.ones((16,), jnp.int32)
    ones16f = jnp.ones((16,), jnp.float32)
    iota16 = lax.iota(jnp.int32, 16)

    for k in range(256 // 16):
        x = (k * 16 + iota16).astype(jnp.float32) * (1.0 / T) + 1e-10
        bits = plsc.bitcast(x, jnp.int32)
        ex = (bits >> 23) - 127
        mant = plsc.bitcast((bits & 0x7FFFFF) | 0x3F800000, jnp.float32)
        adj = mant >= 1.4142135
        mant = jnp.where(adj, 0.5 * mant, mant)
        ef = ex.astype(jnp.float32) + jnp.where(adj, 1.0, 0.0)
        t = (mant - 1.0) / (mant + 1.0)
        t2 = t * t
        lnm = t * (2.0 + t2 * (2.0 / 3.0 + t2 * (2.0 / 5.0 + t2 * (2.0 / 7.0))))
        lnx = ef * 0.69314718 + lnm
        lut_v[pl.ds(16 * k, 16)] = lnx * (-1.0 / T)

    for k in range(NWAY * RH // 16):
        rowhist_v[pl.ds(16 * k, 16)] = zeros16i
    for k in range(GH // 16):
        tothist_v[pl.ds(16 * k, 16)] = jnp.zeros((16,), jnp.float32)

    mlast = iota16 >= (16 * NCHUNK - T)

    def group_body(p, carry):
        iis = [NWAY * p + w for w in range(NWAY)]
        for t in range(NCHUNK):
            off = 16 * t if t < NCHUNK - 1 else T - 16
            m = mlast if t == NCHUNK - 1 else None
            codes = []
            for w, i in enumerate(iis):
                a = ids_v[i, pl.ds(off, 16)]
                b = ids_v[i, pl.ds(off + 1, 16)]
                codes.append(a * A + b + w * RH)
            for w in range(NWAY):
                codes_v[pl.ds(16 * (w * NCHUNK + t), 16)] = codes[w]
            for w in range(NWAY):
                plsc.addupdate_scatter(rowhist_v, [codes[w]], ones16i,
                                       mask=m)
            for w in range(NWAY):
                plsc.addupdate_scatter(tothist_v, [codes[w] - w * RH],
                                       ones16f, mask=m)
        accs = [jnp.zeros((16,), jnp.float32) for _ in range(NWAY)]
        for t in range(NCHUNK):
            m = mlast if t == NCHUNK - 1 else None
            codes = [codes_v[pl.ds(16 * (w * NCHUNK + t), 16)]
                     for w in range(NWAY)]
            cnts = [plsc.load_gather(rowhist_v, [codes[w]], mask=m)
                    for w in range(NWAY)]
            lvs = [plsc.load_gather(lut_v, [cnts[w]], mask=m)
                   for w in range(NWAY)]
            if m is not None:
                lvs = [jnp.where(m, lv, 0.0) for lv in lvs]
            accs = [acc + lv for acc, lv in zip(accs, lvs)]
        for k in range(NWAY * RH // 16):
            rowhist_v[pl.ds(16 * k, 16)] = zeros16i
        for w, i in enumerate(iis):
            plsc.store_scatter(me_v, [jnp.full((16,), i, jnp.int32)],
                               jnp.full((16,), jnp.sum(accs[w]), jnp.float32),
                               mask=iota16 == 0)
        return carry

    lax.fori_loop(0, rows_per // NWAY, group_body, 0)

    pltpu.sync_copy(me_v, me_hbm.at[pl.ds(base, rows_per)])
    pltpu.sync_copy(me_v, me2_hbm.at[wid])
    pltpu.sync_copy(tothist_v, gh_hbm.at[wid])


def _sc_call(ids):
    info = plsc.get_sparse_core_info()
    nc, ns = info.num_cores, info.num_subcores
    nw = nc * ns
    rows_per = B // nw
    mesh = plsc.VectorSubcoreMesh(core_axis_name="c", subcore_axis_name="s")
    fn = pl.kernel(
        functools.partial(_sc_body, rows_per, nc),
        out_type=[
            jax.ShapeDtypeStruct((B,), jnp.float32),
            jax.ShapeDtypeStruct((nw, rows_per), jnp.float32),
            jax.ShapeDtypeStruct((nw, GH), jnp.float32),
        ],
        mesh=mesh,
        compiler_params=pltpu.CompilerParams(needs_layout_passes=False),
        scratch_types=[
            pltpu.VMEM((rows_per, S), jnp.int32),
            pltpu.VMEM((256,), jnp.float32),
            pltpu.VMEM((NWAY * 16 * NCHUNK,), jnp.int32),
            pltpu.VMEM((NWAY * RH,), jnp.int32),
            pltpu.VMEM((GH,), jnp.float32),
            pltpu.VMEM((rows_per,), jnp.float32),
        ],
    )
    return fn(ids)


def _graph_body(nw, hists_ref, me_ref, embed_ref, w1_ref, asrc1_ref,
                adst1_ref, b1_ref, w2_ref, asrc2_ref, adst2_ref, b2_ref,
                wr1_ref, br1_ref, wr2_ref, br2_ref, we_ref, be_ref,
                wotop_ref, wobot_ref, bo_ref, out_ref):
    f32 = jnp.float32

    gh = hists_ref[0:1, :]
    for n in range(1, nw):
        gh = gh + hists_ref[n:n + 1, :]
    ghm = jnp.concatenate([gh[0:1, A * s:A * s + A] for s in range(A)],
                          axis=0)
    ri = lax.broadcasted_iota(jnp.int32, (A, A), 0)
    ci = lax.broadcasted_iota(jnp.int32, (A, A), 1)
    mask = (ghm > 0.0) | (ri == ci)

    def gat_attention(asrc_col, adst_row, hsrc):
        f = asrc_col + adst_row
        f = jnp.where(f >= 0.0, f, 0.2 * f)
        f = jnp.where(mask, f, -1e9)
        m = jnp.max(f, axis=0, keepdims=True)
        p = jnp.exp(f - m)
        att = p / jnp.sum(p, axis=0, keepdims=True)
        return lax.dot_general(att, hsrc, (((0,), (0,)), ((), ())),
                               preferred_element_type=f32)

    dn_t = (((1,), (1,)), ((), ()))

    x = embed_ref[...]
    h1 = jnp.dot(x, w1_ref[...], preferred_element_type=f32)
    heads = []
    for h in range(4):
        hh = h1[:, 64 * h:64 * h + 64]
        asrc = lax.dot_general(hh, asrc1_ref[h:h + 1, :], dn_t,
                               preferred_element_type=f32)
        adst = lax.dot_general(adst1_ref[h:h + 1, :], hh, dn_t,
                               preferred_element_type=f32)
        heads.append(gat_attention(asrc, adst, hh))
    h1o = jnp.concatenate(heads, axis=1) + b1_ref[...]
    h1o = jnp.where(h1o > 0.0, h1o, jnp.exp(h1o) - 1.0)

    h2 = jnp.dot(h1o, w2_ref[...], preferred_element_type=f32)
    asrc2 = lax.dot_general(h2, asrc2_ref[...], dn_t,
                            preferred_element_type=f32)
    adst2 = lax.dot_general(adst2_ref[...], h2, dn_t,
                            preferred_element_type=f32)
    h2o = gat_attention(asrc2, adst2, h2) + b2_ref[...]

    hflat = jnp.concatenate([h2o[n:n + 1, :] for n in range(A)], axis=1)
    g1 = jnp.dot(hflat, wr1_ref[...], preferred_element_type=f32)
    g1 = g1 + br1_ref[...]
    g1 = jnp.maximum(g1, 0.0)
    g2 = jnp.dot(g1, wr2_ref[...], preferred_element_type=f32) + br2_ref[...]

    const = (jnp.dot(g2, wotop_ref[...], preferred_element_type=f32)
             + jnp.dot(be_ref[...], wobot_ref[...],
                       preferred_element_type=f32)
             + bo_ref[...])
    v = jnp.dot(we_ref[...], wobot_ref[...],
                preferred_element_type=f32)

    dn_outer = (((0,), (0,)), ((), ()))
    nblk = me_ref.shape[0]
    rows = me_ref.shape[1]
    for r in range(nblk):
        mev = lax.dot_general(me_ref[r:r + 1, :], v, dn_outer,
                              preferred_element_type=f32)
        out_ref[pl.ds(rows * r, rows), :] = const + mev


def _graph_call(hists3, me2, embed, w1, a_src1, a_dst1, b1r, w2, a_src2,
                a_dst2, b2r, wr1, br1r, wr2, br2r, we, ber, wotop, wobot,
                bor):
    nw = hists3.shape[0]
    return pl.pallas_call(
        functools.partial(_graph_body, nw),
        out_shape=jax.ShapeDtypeStruct((B, D), jnp.float32),
    )(hists3, me2, embed, w1, a_src1, a_dst1, b1r, w2, a_src2, a_dst2, b2r,
      wr1, br1r, wr2, br2r, we, ber, wotop, wobot, bor)


def kernel(activity_ids, embed, W1, a_src1, a_dst1, b1, W2, a_src2, a_dst2,
           b2, Wr1, br1, Wr2, br2, We, be, Wo, bo):
    me, me2, hists = _sc_call(activity_ids)
    h_mp = _graph_call(
        hists, me2, embed, W1,
        a_src1, a_dst1, b1.reshape(1, 256), W2, a_src2, a_dst2,
        b2.reshape(1, D), Wr1, br1.reshape(1, 256), Wr2, br2.reshape(1, D),
        We, be.reshape(1, 32), Wo[:D], Wo[D:], bo.reshape(1, D))
    return h_mp, me
